# baseline (speedup 1.0000x reference)
import dataclasses
import functools

import jax
import jax.numpy as jnp
from jax.experimental import pallas as pl
from jax.experimental.pallas import tpu as pltpu
from jax.experimental.pallas import tpu_sc as plsc

NUM_EMBEDDINGS = 512
EMBEDDING_DIM = 32
BETA = 0.25
ROW_TILE = 2048
N_CHUNKS = 4
N_WORKERS = 32
SC_LANES = 16
GATHER_CHUNK = 128


def _vq_body(z_ref, cb_ref, cn_ref, idx_ref, dsum_ref, dsum_acc, *, n_tiles):
    i = pl.program_id(0)
    z = z_ref[...]
    cb = cb_ref[...]
    dot2 = jax.lax.dot_general(cb, z + z, (((0,), (1,)), ((), ())),
                               preferred_element_type=jnp.float32)
    zz = z * z
    znorm = jnp.sum(zz.T, axis=0, keepdims=True)
    d = znorm - dot2 + cn_ref[...]
    dmin = jnp.min(d, axis=0, keepdims=True)
    k_iota = jax.lax.broadcasted_iota(jnp.int32, d.shape, 0).astype(jnp.float32)
    idxf = jnp.min(jnp.where(d == dmin, k_iota, float(NUM_EMBEDDINGS)),
                   axis=0, keepdims=True)
    idx_ref[0, :, :] = idxf.astype(jnp.int32)

    @pl.when(i == 0)
    def _init():
        dsum_acc[...] = jnp.zeros_like(dsum_acc)

    dsum_acc[...] += dmin

    @pl.when(i == n_tiles - 1)
    def _emit():
        dsum_ref[...] = jnp.full((1, 1), jnp.sum(dsum_acc[...]), jnp.float32)


def _tc_chunk(z, codebook, cnorm, n_tiles):
    body = functools.partial(_vq_body, n_tiles=n_tiles)
    return pl.pallas_call(
        body,
        grid=(n_tiles,),
        in_specs=[
            pl.BlockSpec((ROW_TILE, EMBEDDING_DIM), lambda i: (i, 0)),
            pl.BlockSpec((EMBEDDING_DIM, NUM_EMBEDDINGS), lambda i: (0, 0)),
            pl.BlockSpec((NUM_EMBEDDINGS, 1), lambda i: (0, 0)),
        ],
        out_specs=[
            pl.BlockSpec((1, 1, ROW_TILE), lambda i: (i, 0, 0)),
            pl.BlockSpec((1, 1), lambda i: (0, 0)),
        ],
        out_shape=[
            jax.ShapeDtypeStruct((n_tiles, 1, ROW_TILE), jnp.int32),
            jax.ShapeDtypeStruct((1, 1), jnp.float32),
        ],
        scratch_shapes=[
            pltpu.VMEM((1, ROW_TILE), jnp.float32),
        ],
        compiler_params=pltpu.CompilerParams(
            dimension_semantics=("arbitrary",),
        ),
    )(z, codebook, cnorm)


def _sc_gather_hist(table, idx3, chunk_rows):
    mesh = plsc.VectorSubcoreMesh(core_axis_name="c", subcore_axis_name="s")
    rows_per_w = chunk_rows // N_WORKERS
    tiles_per_w = rows_per_w // ROW_TILE if rows_per_w >= ROW_TILE else 0
    assert rows_per_w % GATHER_CHUNK == 0
    n_sub_chunks = rows_per_w // GATHER_CHUNK
    cp = pltpu.CompilerParams()
    if "needs_layout_passes" in pltpu.CompilerParams.__dataclass_fields__:
        cp = dataclasses.replace(cp, needs_layout_passes=False)

    workers_per_tile = ROW_TILE // rows_per_w

    @functools.partial(
        pl.kernel,
        mesh=mesh,
        compiler_params=cp,
        out_type=[
            jax.ShapeDtypeStruct((chunk_rows, EMBEDDING_DIM), jnp.float32),
            jax.ShapeDtypeStruct((N_WORKERS, NUM_EMBEDDINGS), jnp.float32),
        ],
        scratch_types=[
            pltpu.VMEM((rows_per_w,), jnp.int32),
            pltpu.VMEM((NUM_EMBEDDINGS, EMBEDDING_DIM), jnp.float32),
            pltpu.VMEM((GATHER_CHUNK, EMBEDDING_DIM), jnp.float32),
            pltpu.VMEM((NUM_EMBEDDINGS,), jnp.float32),
        ],
    )
    def sc_kernel(table_hbm, idx_hbm, out_hbm, counts_hbm,
                  idx_v, table_v, rows_v, counts_v):
        wid = jax.lax.axis_index("s") * 2 + jax.lax.axis_index("c")
        base = wid * rows_per_w
        pltpu.sync_copy(table_hbm, table_v)
        tile_id = wid // workers_per_tile
        tile_off = (wid % workers_per_tile) * rows_per_w
        pltpu.sync_copy(
            idx_hbm.at[tile_id, 0, pl.ds(tile_off, rows_per_w)], idx_v)

        @pl.loop(0, NUM_EMBEDDINGS, step=SC_LANES)
        def _zero(j):
            counts_v[pl.ds(j, SC_LANES)] = jnp.zeros((SC_LANES,), jnp.float32)

        ones = jnp.ones((SC_LANES,), jnp.float32)
        lane_iota = jax.lax.iota(jnp.int32, SC_LANES)

        @pl.loop(0, rows_per_w, step=GATHER_CHUNK)
        def _chunk(c0):
            for gg in range(GATHER_CHUNK // SC_LANES):
                g = c0 + gg * SC_LANES
                iv = idx_v[pl.ds(g, SC_LANES)]
                plsc.addupdate_scatter(counts_v, [iv], ones)
                pos = lane_iota + gg * SC_LANES
                for c in range(EMBEDDING_DIM):
                    col = jnp.full((SC_LANES,), c, jnp.int32)
                    vals = plsc.load_gather(table_v, [iv, col])
                    plsc.store_scatter(rows_v, [pos, col], vals)
            pltpu.sync_copy(rows_v,
                            out_hbm.at[pl.ds(base + c0, GATHER_CHUNK)])

        pltpu.sync_copy(counts_v, counts_hbm.at[wid])

    return sc_kernel(table, idx3)


def _final_body(counts_ref, dsum_ref, perp_ref, loss_ref, *, n_rows):
    counts = jnp.sum(counts_ref[...], axis=(0, 1)).reshape(1, NUM_EMBEDDINGS)
    avg = counts * (1.0 / n_rows)
    perp = jnp.exp(-jnp.sum(avg * jnp.log(avg + 1e-10)))
    perp_ref[...] = jnp.full((1, 1), perp, jnp.float32)
    total = jnp.sum(dsum_ref[...])
    loss_ref[...] = jnp.full((1, 1),
                             total * (BETA / (n_rows * EMBEDDING_DIM)),
                             jnp.float32)


def kernel(inputs, codebook, ema_cs_hidden, ema_dw_hidden, counter, training):
    batch, hw, dim = inputs.shape
    n_rows = batch * hw
    chunk_rows = n_rows // N_CHUNKS
    tiles_per_chunk = chunk_rows // ROW_TILE
    z = inputs.reshape(N_CHUNKS, chunk_rows, dim)
    cnorm = jnp.sum(codebook * codebook, axis=0).reshape(NUM_EMBEDDINGS, 1)
    table = codebook.T

    ste_parts, counts_parts, dsum_parts = [], [], []
    for ci in range(N_CHUNKS):
        idx3, dsum1 = _tc_chunk(z[ci], codebook, cnorm, tiles_per_chunk)
        ste_c, counts_c = _sc_gather_hist(table, idx3, chunk_rows)
        ste_parts.append(ste_c)
        counts_parts.append(counts_c)
        dsum_parts.append(dsum1)

    counts_all = jnp.stack(counts_parts)
    dsum_all = jnp.concatenate(dsum_parts, axis=0)

    perp2, loss2 = pl.pallas_call(
        functools.partial(_final_body, n_rows=n_rows),
        in_specs=[
            pl.BlockSpec((N_CHUNKS, N_WORKERS, NUM_EMBEDDINGS),
                         lambda: (0, 0, 0)),
            pl.BlockSpec((N_CHUNKS, 1), lambda: (0, 0)),
        ],
        out_specs=[
            pl.BlockSpec((1, 1), lambda: (0, 0)),
            pl.BlockSpec((1, 1), lambda: (0, 0)),
        ],
        out_shape=[
            jax.ShapeDtypeStruct((1, 1), jnp.float32),
            jax.ShapeDtypeStruct((1, 1), jnp.float32),
        ],
    )(counts_all, dsum_all)

    ste = jnp.concatenate(ste_parts, axis=0).reshape(batch, hw, dim)
    return ste, perp2.reshape(()), loss2.reshape(())


# speedup vs baseline: 2.2363x; 2.2363x over previous
import functools

import jax
import jax.numpy as jnp
from jax.experimental import pallas as pl
from jax.experimental.pallas import tpu as pltpu

NUM_EMBEDDINGS = 512
EMBEDDING_DIM = 32
BETA = 0.25
ROW_TILE = 8192


def _vq_body(z_ref, cb_ref, cn_ref, counts_ref, dsum_ref, q_ref):
    z = z_ref[...]
    cb = cb_ref[...]
    dot2 = jnp.dot(z + z, cb, preferred_element_type=jnp.float32)
    znorm = jnp.sum(z * z, axis=1, keepdims=True)
    d = znorm - dot2 + cn_ref[...]
    dmin = jnp.min(d, axis=1, keepdims=True)
    k_iota = jax.lax.broadcasted_iota(
        jnp.int32, (1, NUM_EMBEDDINGS), 1).astype(jnp.float32)
    idxf = jnp.min(jnp.where(d == dmin, k_iota, float(NUM_EMBEDDINGS)),
                   axis=1, keepdims=True)
    onehot = (k_iota == idxf).astype(jnp.float32)
    counts_ref[0, 0, :] = jnp.sum(onehot, axis=0)
    dsum_ref[0, 0, :] = jnp.full((128,), jnp.sum(dmin), jnp.float32)
    q_ref[...] = jax.lax.dot_general(onehot, cb, (((1,), (1,)), ((), ())),
                                     preferred_element_type=jnp.float32)


def _finalize_body(counts_ref, dsum_ref, perp_ref, loss_ref, *, n_rows):
    counts = jnp.sum(counts_ref[...], axis=(0, 1)).reshape(1, NUM_EMBEDDINGS)
    avg = counts * (1.0 / n_rows)
    perp = jnp.exp(-jnp.sum(avg * jnp.log(avg + 1e-10)))
    perp_ref[...] = jnp.full((1, 1), perp, jnp.float32)
    total = jnp.sum(dsum_ref[:, :, 0])
    loss_ref[...] = jnp.full((1, 1),
                             total * (BETA / (n_rows * EMBEDDING_DIM)),
                             jnp.float32)


def kernel(inputs, codebook, ema_cs_hidden, ema_dw_hidden, counter, training):
    batch, hw, dim = inputs.shape
    n_rows = batch * hw
    n_tiles = n_rows // ROW_TILE
    z = inputs.reshape(n_rows, dim)
    cnorm = jnp.sum(codebook * codebook, axis=0, keepdims=True)

    counts3, dsum3, q = pl.pallas_call(
        _vq_body,
        grid=(n_tiles,),
        in_specs=[
            pl.BlockSpec((ROW_TILE, dim), lambda i: (i, 0)),
            pl.BlockSpec((dim, NUM_EMBEDDINGS), lambda i: (0, 0)),
            pl.BlockSpec((1, NUM_EMBEDDINGS), lambda i: (0, 0)),
        ],
        out_specs=[
            pl.BlockSpec((1, 1, NUM_EMBEDDINGS), lambda i: (i, 0, 0)),
            pl.BlockSpec((1, 1, 128), lambda i: (i, 0, 0)),
            pl.BlockSpec((ROW_TILE, dim), lambda i: (i, 0)),
        ],
        out_shape=[
            jax.ShapeDtypeStruct((n_tiles, 1, NUM_EMBEDDINGS), jnp.float32),
            jax.ShapeDtypeStruct((n_tiles, 1, 128), jnp.float32),
            jax.ShapeDtypeStruct((n_rows, dim), jnp.float32),
        ],
        compiler_params=pltpu.CompilerParams(
            dimension_semantics=("parallel",),
        ),
    )(z, codebook, cnorm)

    perp2, loss2 = pl.pallas_call(
        functools.partial(_finalize_body, n_rows=n_rows),
        in_specs=[
            pl.BlockSpec((n_tiles, 1, NUM_EMBEDDINGS), lambda: (0, 0, 0)),
            pl.BlockSpec((n_tiles, 1, 128), lambda: (0, 0, 0)),
        ],
        out_specs=[
            pl.BlockSpec((1, 1), lambda: (0, 0)),
            pl.BlockSpec((1, 1), lambda: (0, 0)),
        ],
        out_shape=[
            jax.ShapeDtypeStruct((1, 1), jnp.float32),
            jax.ShapeDtypeStruct((1, 1), jnp.float32),
        ],
    )(counts3, dsum3)

    ste = q.reshape(batch, hw, dim)
    return ste, perp2.reshape(()), loss2.reshape(())


# speedup vs baseline: 2.2544x; 1.0081x over previous
import functools

import jax
import jax.numpy as jnp
from jax.experimental import pallas as pl
from jax.experimental.pallas import tpu as pltpu

NUM_EMBEDDINGS = 512
EMBEDDING_DIM = 32
BETA = 0.25
ROW_TILE = 8192


def _vq_body(z_ref, cb_ref, cn_ref, counts_ref, dsum_ref, q_ref):
    z = z_ref[...]
    cb = cb_ref[...]
    dot2 = jnp.dot(z + z, cb, preferred_element_type=jnp.float32)
    znorm = jnp.sum(z * z, axis=1, keepdims=True)
    d = znorm - dot2 + cn_ref[...]
    dmin = jnp.min(d, axis=1, keepdims=True)
    k_iota = jax.lax.broadcasted_iota(
        jnp.int32, (1, NUM_EMBEDDINGS), 1).astype(jnp.float32)
    idxf = jnp.min(jnp.where(d == dmin, k_iota, float(NUM_EMBEDDINGS)),
                   axis=1, keepdims=True)
    mask = k_iota == idxf
    counts_ref[0, 0, :] = jnp.sum(mask.astype(jnp.float32), axis=0)
    dsum_ref[0, 0, :] = jnp.full((128,), jnp.sum(dmin), jnp.float32)
    q_ref[...] = jax.lax.dot_general(
        mask.astype(jnp.bfloat16), cb.astype(jnp.bfloat16),
        (((1,), (1,)), ((), ())), preferred_element_type=jnp.float32)


def _finalize_body(counts_ref, dsum_ref, perp_ref, loss_ref, *, n_rows):
    counts = jnp.sum(counts_ref[...], axis=(0, 1)).reshape(1, NUM_EMBEDDINGS)
    avg = counts * (1.0 / n_rows)
    perp = jnp.exp(-jnp.sum(avg * jnp.log(avg + 1e-10)))
    perp_ref[...] = jnp.full((1, 1), perp, jnp.float32)
    total = jnp.sum(dsum_ref[:, :, 0])
    loss_ref[...] = jnp.full((1, 1),
                             total * (BETA / (n_rows * EMBEDDING_DIM)),
                             jnp.float32)


def kernel(inputs, codebook, ema_cs_hidden, ema_dw_hidden, counter, training):
    batch, hw, dim = inputs.shape
    n_rows = batch * hw
    n_tiles = n_rows // ROW_TILE
    z = inputs.reshape(n_rows, dim)
    cnorm = jnp.sum(codebook * codebook, axis=0, keepdims=True)

    counts3, dsum3, q = pl.pallas_call(
        _vq_body,
        grid=(n_tiles,),
        in_specs=[
            pl.BlockSpec((ROW_TILE, dim), lambda i: (i, 0)),
            pl.BlockSpec((dim, NUM_EMBEDDINGS), lambda i: (0, 0)),
            pl.BlockSpec((1, NUM_EMBEDDINGS), lambda i: (0, 0)),
        ],
        out_specs=[
            pl.BlockSpec((1, 1, NUM_EMBEDDINGS), lambda i: (i, 0, 0)),
            pl.BlockSpec((1, 1, 128), lambda i: (i, 0, 0)),
            pl.BlockSpec((ROW_TILE, dim), lambda i: (i, 0)),
        ],
        out_shape=[
            jax.ShapeDtypeStruct((n_tiles, 1, NUM_EMBEDDINGS), jnp.float32),
            jax.ShapeDtypeStruct((n_tiles, 1, 128), jnp.float32),
            jax.ShapeDtypeStruct((n_rows, dim), jnp.float32),
        ],
        compiler_params=pltpu.CompilerParams(
            dimension_semantics=("parallel",),
        ),
    )(z, codebook, cnorm)

    perp2, loss2 = pl.pallas_call(
        functools.partial(_finalize_body, n_rows=n_rows),
        in_specs=[
            pl.BlockSpec((n_tiles, 1, NUM_EMBEDDINGS), lambda: (0, 0, 0)),
            pl.BlockSpec((n_tiles, 1, 128), lambda: (0, 0, 0)),
        ],
        out_specs=[
            pl.BlockSpec((1, 1), lambda: (0, 0)),
            pl.BlockSpec((1, 1), lambda: (0, 0)),
        ],
        out_shape=[
            jax.ShapeDtypeStruct((1, 1), jnp.float32),
            jax.ShapeDtypeStruct((1, 1), jnp.float32),
        ],
    )(counts3, dsum3)

    ste = q.reshape(batch, hw, dim)
    return ste, perp2.reshape(()), loss2.reshape(())
